# baseline (speedup 1.0000x reference)
import functools
import jax
import jax.numpy as jnp
from jax.experimental import pallas as pl
from jax.experimental.pallas import tpu as pltpu

B, N, K = 4, 10000, 17
KN = K - 1
NT = B * N
D = 128
EPS = 1e-5
ROWS_BLK = 2000
N_BLKS = NT // ROWS_BLK


def _main_body(x_ref, accx_ref, hist_ref, w_ref, b_ref, z_ref, stats_ref):
    i = pl.program_id(0)
    hist = hist_ref[...].astype(jnp.float32)
    dinv = jax.lax.rsqrt(1.0 + hist)
    xb = x_ref[...]
    accx = accx_ref[...]
    y = dinv * accx + (dinv * dinv) * xb
    z = jnp.dot(y, w_ref[...], preferred_element_type=jnp.float32)
    z = z + b_ref[0, :][None, :]
    z_ref[...] = z

    @pl.when(i == 0)
    def _():
        stats_ref[...] = jnp.zeros_like(stats_ref)

    s = jnp.sum(z, axis=0)
    s2 = jnp.sum(z * z, axis=0)
    stats_ref[0, :] += s
    stats_ref[1, :] += s2


def _finish_body(z_ref, stats_ref, g_ref, beta_ref, a_ref, o_ref):
    m = float(NT * D)
    mu = jnp.sum(stats_ref[0, :]) / m
    var = jnp.sum(stats_ref[1, :]) / m - mu * mu
    rstd = jax.lax.rsqrt(var + EPS)
    z = z_ref[...]
    h = (z - mu) * rstd * g_ref[0, :][None, :] + beta_ref[0, :][None, :]
    a = a_ref[0, 0]
    o_ref[...] = jnp.where(h >= 0, h, a * h)


def _dense_part(x2, accx, hist, W, b, gamma, beta, prelu_a):
    z, stats = pl.pallas_call(
        _main_body,
        grid=(N_BLKS,),
        in_specs=[
            pl.BlockSpec((ROWS_BLK, D), lambda i: (i, 0)),
            pl.BlockSpec((ROWS_BLK, D), lambda i: (i, 0)),
            pl.BlockSpec((ROWS_BLK, 1), lambda i: (i, 0)),
            pl.BlockSpec((D, D), lambda i: (0, 0)),
            pl.BlockSpec((1, D), lambda i: (0, 0)),
        ],
        out_specs=[
            pl.BlockSpec((ROWS_BLK, D), lambda i: (i, 0)),
            pl.BlockSpec((8, D), lambda i: (0, 0)),
        ],
        out_shape=[
            jax.ShapeDtypeStruct((NT, D), jnp.float32),
            jax.ShapeDtypeStruct((8, D), jnp.float32),
        ],
    )(x2, accx, hist.reshape(NT, 1), W, b.reshape(1, D))

    out = pl.pallas_call(
        _finish_body,
        grid=(N_BLKS,),
        in_specs=[
            pl.BlockSpec((ROWS_BLK, D), lambda i: (i, 0)),
            pl.BlockSpec((8, D), lambda i: (0, 0)),
            pl.BlockSpec((1, D), lambda i: (0, 0)),
            pl.BlockSpec((1, D), lambda i: (0, 0)),
            pl.BlockSpec((1, 1), lambda i: (0, 0)),
        ],
        out_specs=pl.BlockSpec((ROWS_BLK, D), lambda i: (i, 0)),
        out_shape=jax.ShapeDtypeStruct((NT, D), jnp.float32),
    )(z, stats, gamma.reshape(1, D), beta.reshape(1, D),
      prelu_a.reshape(1, 1))
    return out


def kernel(x, edge_index, edge_mask, W, b, gamma, beta, prelu_a):
    x2 = x[:, 0, :]
    ei = edge_index[:, :, 1:].astype(jnp.int32)
    mask = edge_mask[:, :, 1:]

    flat_pos = jnp.arange(N * KN, dtype=jnp.int32).reshape(1, N, KN)
    dump = N + (flat_pos % 128)
    dst = jnp.where(mask, ei, dump)

    R = N + 128
    dstf = dst.reshape(B, N * KN)
    hist_b = jnp.zeros((B, R), jnp.int32).at[
        jnp.arange(B)[:, None], dstf].add(1, mode="drop")
    hist = hist_b[:, :N].reshape(NT)
    dinv = jax.lax.rsqrt(1.0 + hist.astype(jnp.float32))
    xs = x2 * dinv[:, None]
    xs_b = xs.reshape(B, N, D)
    acc = jnp.zeros((B, R, D), jnp.float32)
    src_rows = jnp.broadcast_to(
        jnp.arange(N)[None, :, None], (B, N, KN))
    acc = acc.at[jnp.arange(B)[:, None, None], dst].add(
        xs_b[jnp.arange(B)[:, None, None], src_rows], mode="drop")
    accx = acc[:, :N].reshape(NT, D)

    return _dense_part(x2, accx, hist, W, b, gamma, beta, prelu_a)


# speedup vs baseline: 39.5681x; 39.5681x over previous
import dataclasses
import functools
import jax
import jax.numpy as jnp
from jax import lax
from jax.experimental import pallas as pl
from jax.experimental.pallas import tpu as pltpu
from jax.experimental.pallas import tpu_sc as plsc

B, N, K = 4, 10000, 17
KN = K - 1
NT = B * N
D = 128
EPS = 1e-5
ROWS_BLK = 2000
N_BLKS = NT // ROWS_BLK


def _main_body(x_ref, accx_ref, hist_ref, w_ref, b_ref, z_ref, stats_ref):
    i = pl.program_id(0)
    hist = hist_ref[...].astype(jnp.float32)
    dinv = jax.lax.rsqrt(1.0 + hist)
    xb = x_ref[...]
    accx = accx_ref[...]
    y = dinv * accx + (dinv * dinv) * xb
    z = jnp.dot(y, w_ref[...], preferred_element_type=jnp.float32)
    z = z + b_ref[0, :][None, :]
    z_ref[...] = z

    @pl.when(i == 0)
    def _():
        stats_ref[...] = jnp.zeros_like(stats_ref)

    s = jnp.sum(z, axis=0)
    s2 = jnp.sum(z * z, axis=0)
    stats_ref[0, :] += s
    stats_ref[1, :] += s2


def _finish_body(z_ref, stats_ref, g_ref, beta_ref, a_ref, o_ref):
    m = float(NT * D)
    mu = jnp.sum(stats_ref[0, :]) / m
    var = jnp.sum(stats_ref[1, :]) / m - mu * mu
    rstd = jax.lax.rsqrt(var + EPS)
    z = z_ref[...]
    h = (z - mu) * rstd * g_ref[0, :][None, :] + beta_ref[0, :][None, :]
    a = a_ref[0, 0]
    o_ref[...] = jnp.where(h >= 0, h, a * h)


def _scale_body(x_ref, hist_ref, xs_ref):
    dinv = jax.lax.rsqrt(1.0 + hist_ref[...].astype(jnp.float32))
    xs_ref[...] = x_ref[...] * dinv


def _scale(x2, hist):
    return pl.pallas_call(
        _scale_body,
        grid=(N_BLKS,),
        in_specs=[
            pl.BlockSpec((ROWS_BLK, D), lambda i: (i, 0)),
            pl.BlockSpec((ROWS_BLK, 1), lambda i: (i, 0)),
        ],
        out_specs=pl.BlockSpec((ROWS_BLK, D), lambda i: (i, 0)),
        out_shape=jax.ShapeDtypeStruct((NT, D), jnp.float32),
    )(x2, hist.reshape(NT, 1))


def _dense_part(x2, accx, hist, W, b, gamma, beta, prelu_a):
    z, stats = pl.pallas_call(
        _main_body,
        grid=(N_BLKS,),
        in_specs=[
            pl.BlockSpec((ROWS_BLK, D), lambda i: (i, 0)),
            pl.BlockSpec((ROWS_BLK, D), lambda i: (i, 0)),
            pl.BlockSpec((ROWS_BLK, 1), lambda i: (i, 0)),
            pl.BlockSpec((D, D), lambda i: (0, 0)),
            pl.BlockSpec((1, D), lambda i: (0, 0)),
        ],
        out_specs=[
            pl.BlockSpec((ROWS_BLK, D), lambda i: (i, 0)),
            pl.BlockSpec((8, D), lambda i: (0, 0)),
        ],
        out_shape=[
            jax.ShapeDtypeStruct((NT, D), jnp.float32),
            jax.ShapeDtypeStruct((8, D), jnp.float32),
        ],
    )(x2, accx, hist.reshape(NT, 1), W, b.reshape(1, D))

    out = pl.pallas_call(
        _finish_body,
        grid=(N_BLKS,),
        in_specs=[
            pl.BlockSpec((ROWS_BLK, D), lambda i: (i, 0)),
            pl.BlockSpec((8, D), lambda i: (0, 0)),
            pl.BlockSpec((1, D), lambda i: (0, 0)),
            pl.BlockSpec((1, D), lambda i: (0, 0)),
            pl.BlockSpec((1, 1), lambda i: (0, 0)),
        ],
        out_specs=pl.BlockSpec((ROWS_BLK, D), lambda i: (i, 0)),
        out_shape=jax.ShapeDtypeStruct((NT, D), jnp.float32),
    )(z, stats, gamma.reshape(1, D), beta.reshape(1, D),
      prelu_a.reshape(1, 1))
    return out


RH = 640
EPT = N * KN // 8


def _sc_params():
    cp = pltpu.CompilerParams()
    if "needs_layout_passes" in pltpu.CompilerParams.__dataclass_fields__:
        cp = dataclasses.replace(cp, needs_layout_passes=False)
    return cp


def _hist_sc(dst):
    mesh = plsc.VectorSubcoreMesh(core_axis_name="c", subcore_axis_name="s")

    @functools.partial(
        pl.kernel,
        out_type=jax.ShapeDtypeStruct((B, RH, 16), jnp.int32),
        mesh=mesh,
        scratch_types=[
            pltpu.VMEM((EPT,), jnp.int32),
            pltpu.VMEM((RH, 16), jnp.int32),
            pltpu.VMEM((RH,), jnp.int32),
            pltpu.VMEM_SHARED((2, RH, 16), jnp.int32),
            pltpu.SemaphoreType.DMA,
        ],
        compiler_params=_sc_params(),
    )
    def hist_kernel(dst_hbm, out_hbm, dstv, histv, idsv, slab, sem):
        c = lax.axis_index("c")
        s = lax.axis_index("s")
        half = s // 8
        j = s % 8
        batch = 2 * c + half
        iota16 = lax.iota(jnp.int32, 16)

        @pl.loop(0, RH)
        def _(i):
            histv[i, :] = jnp.zeros((16,), jnp.int32)

        @pl.loop(0, RH // 16)
        def _(i):
            idsv[pl.ds(i * 16, 16)] = iota16 + i * 16

        @pl.when(j == 0)
        def _():
            pltpu.sync_copy(histv, slab.at[half])

        e0 = pl.multiple_of((batch * 8 + j) * EPT, 8)
        pltpu.async_copy(dst_hbm.at[pl.ds(e0, EPT)], dstv, sem).wait()
        plsc.subcore_barrier()

        ones16 = jnp.ones((16,), jnp.int32)

        @pl.loop(0, EPT // 16)
        def _(i):
            d = dstv[pl.ds(i * 16, 16)]
            plsc.addupdate_scatter(histv, [d >> 4, d & 15], ones16)

        pltpu.sync_copy(histv, slab.at[half].at[idsv], add=True)
        plsc.subcore_barrier()

        rows = RH // 8
        r0 = pl.multiple_of(j * rows, 8)
        pltpu.sync_copy(slab.at[half, pl.ds(r0, rows)],
                        out_hbm.at[batch, pl.ds(r0, rows)])

    return hist_kernel(dst)


RS = 10240
SPT = N // 16
SPAD = 640


CHA = 64
NCHA = SPAD // CHA


def _agg_sc(xs_tiled, dst_flat):
    mesh = plsc.VectorSubcoreMesh(core_axis_name="c", subcore_axis_name="s")

    @functools.partial(
        pl.kernel,
        out_type=jax.ShapeDtypeStruct((B, RS, D), jnp.float32),
        mesh=mesh,
        scratch_types=[
            pltpu.VMEM((CHA, D), jnp.float32),
            pltpu.VMEM((CHA, D), jnp.float32),
            pltpu.VMEM_SHARED((RS, D), jnp.float32),
            pltpu.SemaphoreType.DMA,
        ] + [pltpu.VMEM((CHA,), jnp.int32) for _ in range(KN)],
        compiler_params=_sc_params(),
    )
    def agg_kernel(xs_hbm, dst_hbm, out_hbm, xsv, zv, slab, sem, *idxs):
        c = lax.axis_index("c")
        t = lax.axis_index("s")
        z16 = jnp.zeros((16,), jnp.float32)

        @pl.loop(0, CHA)
        def _(i):
            @pl.loop(0, D // 16)
            def _(k):
                zv[i, pl.ds(k * 16, 16)] = z16

        @pl.loop(0, 2)
        def _(q):
            batch = 2 * c + q
            w = batch * 16 + t

            @pl.loop(0, RS // 16 // CHA)
            def _(zb):
                r0 = pl.multiple_of(t * (RS // 16) + zb * CHA, CHA)
                pltpu.sync_copy(zv, slab.at[pl.ds(r0, CHA)])

            plsc.subcore_barrier()

            @pl.loop(0, NCHA)
            def _(ch):
                cps = [pltpu.async_copy(
                    xs_hbm.at[w, pl.ds(ch * CHA, CHA)], xsv, sem)]
                for k in range(KN):
                    e0 = pl.multiple_of(
                        (w * KN + k) * SPAD + ch * CHA, 8)
                    cps.append(pltpu.async_copy(
                        dst_hbm.at[pl.ds(e0, CHA)], idxs[k], sem))
                for cp in cps:
                    cp.wait()
                for k in range(KN):
                    pltpu.sync_copy(xsv, slab.at[idxs[k]], add=True)

            plsc.subcore_barrier()

            rows = RS // 16
            r0 = pl.multiple_of(t * rows, 8)
            pltpu.sync_copy(slab.at[pl.ds(r0, rows)],
                            out_hbm.at[batch, pl.ds(r0, rows)])
            plsc.subcore_barrier()

    return agg_kernel(xs_tiled, dst_flat)


def kernel(x, edge_index, edge_mask, W, b, gamma, beta, prelu_a):
    x2 = x[:, 0, :]
    ei = edge_index[:, :, 1:].astype(jnp.int32)
    mask = edge_mask[:, :, 1:]

    flat_pos = jnp.arange(N * KN, dtype=jnp.int32).reshape(1, N, KN)
    dump = N + (flat_pos % 128)
    dst = jnp.where(mask, ei, dump)

    dstf = dst.reshape(B, N * KN)
    hist_b = _hist_sc(dstf.reshape(-1)).reshape(B, RH * 16)
    hist = hist_b[:, :N].reshape(NT)

    xs = _scale(x2, hist)
    xs_tiled = jnp.pad(xs.reshape(B * 16, SPT, D),
                       ((0, 0), (0, SPAD - SPT), (0, 0)))
    dst_tile = jnp.pad(
        dst.reshape(B * 16, SPT, KN).transpose(0, 2, 1),
        ((0, 0), (0, 0), (0, SPAD - SPT)),
        constant_values=N)
    acc = _agg_sc(xs_tiled, dst_tile.reshape(-1))
    accx = acc[:, :N].reshape(NT, D)

    return _dense_part(x2, accx, hist, W, b, gamma, beta, prelu_a)


# speedup vs baseline: 42.5732x; 1.0759x over previous
import dataclasses
import functools
import jax
import jax.numpy as jnp
from jax import lax
from jax.experimental import pallas as pl
from jax.experimental.pallas import tpu as pltpu
from jax.experimental.pallas import tpu_sc as plsc

B, N, K = 4, 10000, 17
KN = K - 1
NT = B * N
D = 128
EPS = 1e-5
ROWS_BLK = 2000
N_BLKS = NT // ROWS_BLK


def _main_body(x_ref, accx_ref, hist_ref, w_ref, b_ref, z_ref, stats_ref):
    i = pl.program_id(0)
    hist = hist_ref[0].astype(jnp.float32)
    dinv = jax.lax.rsqrt(1.0 + hist)
    xb = x_ref[0]
    accx = accx_ref[0]
    y = dinv * accx + (dinv * dinv) * xb
    z = jnp.dot(y, w_ref[...], preferred_element_type=jnp.float32)
    z = z + b_ref[0, :][None, :]
    z_ref[0] = z

    @pl.when(i == 0)
    def _():
        stats_ref[...] = jnp.zeros_like(stats_ref)

    s = jnp.sum(z, axis=0)
    s2 = jnp.sum(z * z, axis=0)
    stats_ref[0, :] += s
    stats_ref[1, :] += s2


def _finish_body(z_ref, stats_ref, g_ref, beta_ref, a_ref, o_ref):
    m = float(NT * D)
    mu = jnp.sum(stats_ref[0, :]) / m
    var = jnp.sum(stats_ref[1, :]) / m - mu * mu
    rstd = jax.lax.rsqrt(var + EPS)
    z = z_ref[0]
    h = (z - mu) * rstd * g_ref[0, :][None, :] + beta_ref[0, :][None, :]
    a = a_ref[0, 0]
    o_ref[0] = jnp.where(h >= 0, h, a * h)


def _scale_body(x_ref, hist_ref, xs_ref):
    dinv = jax.lax.rsqrt(1.0 + hist_ref[...].astype(jnp.float32))
    xs_ref[...] = x_ref[...] * dinv


def _scale(x2, hist):
    return pl.pallas_call(
        _scale_body,
        grid=(N_BLKS,),
        in_specs=[
            pl.BlockSpec((ROWS_BLK, D), lambda i: (i, 0)),
            pl.BlockSpec((ROWS_BLK, 1), lambda i: (i, 0)),
        ],
        out_specs=pl.BlockSpec((ROWS_BLK, D), lambda i: (i, 0)),
        out_shape=jax.ShapeDtypeStruct((NT, D), jnp.float32),
    )(x2, hist.reshape(NT, 1))


def _dense_part(x3, acc, hist3, W, b, gamma, beta, prelu_a):
    NB = N // ROWS_BLK
    z, stats = pl.pallas_call(
        _main_body,
        grid=(B * NB,),
        in_specs=[
            pl.BlockSpec((1, ROWS_BLK, D), lambda i: (i // NB, i % NB, 0)),
            pl.BlockSpec((1, ROWS_BLK, D), lambda i: (i // NB, i % NB, 0)),
            pl.BlockSpec((1, ROWS_BLK, 1), lambda i: (i // NB, i % NB, 0)),
            pl.BlockSpec((D, D), lambda i: (0, 0)),
            pl.BlockSpec((1, D), lambda i: (0, 0)),
        ],
        out_specs=[
            pl.BlockSpec((1, ROWS_BLK, D), lambda i: (i // NB, i % NB, 0)),
            pl.BlockSpec((8, D), lambda i: (0, 0)),
        ],
        out_shape=[
            jax.ShapeDtypeStruct((B, N, D), jnp.float32),
            jax.ShapeDtypeStruct((8, D), jnp.float32),
        ],
    )(x3, acc, hist3, W, b.reshape(1, D))

    out = pl.pallas_call(
        _finish_body,
        grid=(B * NB,),
        in_specs=[
            pl.BlockSpec((1, ROWS_BLK, D), lambda i: (i // NB, i % NB, 0)),
            pl.BlockSpec((8, D), lambda i: (0, 0)),
            pl.BlockSpec((1, D), lambda i: (0, 0)),
            pl.BlockSpec((1, D), lambda i: (0, 0)),
            pl.BlockSpec((1, 1), lambda i: (0, 0)),
        ],
        out_specs=pl.BlockSpec((1, ROWS_BLK, D), lambda i: (i // NB, i % NB, 0)),
        out_shape=jax.ShapeDtypeStruct((B, N, D), jnp.float32),
    )(z, stats, gamma.reshape(1, D), beta.reshape(1, D),
      prelu_a.reshape(1, 1))
    return out


RH = 640
EPT = N * KN // 8


def _sc_params():
    cp = pltpu.CompilerParams()
    if "needs_layout_passes" in pltpu.CompilerParams.__dataclass_fields__:
        cp = dataclasses.replace(cp, needs_layout_passes=False)
    return cp


def _hist_sc(dst):
    mesh = plsc.VectorSubcoreMesh(core_axis_name="c", subcore_axis_name="s")

    @functools.partial(
        pl.kernel,
        out_type=jax.ShapeDtypeStruct((B, RH, 16), jnp.int32),
        mesh=mesh,
        scratch_types=[
            pltpu.VMEM((EPT,), jnp.int32),
            pltpu.VMEM((RH, 16), jnp.int32),
            pltpu.VMEM((RH,), jnp.int32),
            pltpu.VMEM_SHARED((2, RH, 16), jnp.int32),
            pltpu.SemaphoreType.DMA,
        ],
        compiler_params=_sc_params(),
    )
    def hist_kernel(dst_hbm, out_hbm, dstv, histv, idsv, slab, sem):
        c = lax.axis_index("c")
        s = lax.axis_index("s")
        half = s // 8
        j = s % 8
        batch = 2 * c + half
        iota16 = lax.iota(jnp.int32, 16)

        @pl.loop(0, RH)
        def _(i):
            histv[i, :] = jnp.zeros((16,), jnp.int32)

        @pl.loop(0, RH // 16)
        def _(i):
            idsv[pl.ds(i * 16, 16)] = iota16 + i * 16

        @pl.when(j == 0)
        def _():
            pltpu.sync_copy(histv, slab.at[half])

        e0 = pl.multiple_of((batch * 8 + j) * EPT, 8)
        pltpu.async_copy(dst_hbm.at[pl.ds(e0, EPT)], dstv, sem).wait()
        plsc.subcore_barrier()

        ones16 = jnp.ones((16,), jnp.int32)

        @pl.loop(0, EPT // 16)
        def _(i):
            d = dstv[pl.ds(i * 16, 16)]
            plsc.addupdate_scatter(histv, [d >> 4, d & 15], ones16)

        pltpu.sync_copy(histv, slab.at[half].at[idsv], add=True)
        plsc.subcore_barrier()

        rows = RH // 8
        r0 = pl.multiple_of(j * rows, 8)
        pltpu.sync_copy(slab.at[half, pl.ds(r0, rows)],
                        out_hbm.at[batch, pl.ds(r0, rows)])

    return hist_kernel(dst)


RS = 10240
SPT = N // 16
SPAD = 640


CHA = 125
CHP = 128
NCHA = SPT // CHA


def _agg_sc(xs_grp, dst_flat):
    mesh = plsc.VectorSubcoreMesh(core_axis_name="c", subcore_axis_name="s")

    @functools.partial(
        pl.kernel,
        out_type=jax.ShapeDtypeStruct((B, RS, D), jnp.float32),
        mesh=mesh,
        scratch_types=[
            pltpu.VMEM((CHP, D), jnp.float32),
            pltpu.VMEM((64, D), jnp.float32),
            pltpu.VMEM_SHARED((RS, D), jnp.float32),
            pltpu.SemaphoreType.DMA,
        ] + [pltpu.VMEM((CHP,), jnp.int32) for _ in range(KN)],
        compiler_params=_sc_params(),
    )
    def agg_kernel(xs_hbm, dst_hbm, out_hbm, xsv, zv, slab, sem, *idxs):
        c = lax.axis_index("c")
        t = lax.axis_index("s")
        z16 = jnp.zeros((16,), jnp.float32)

        @pl.loop(0, 64)
        def _(i):
            @pl.loop(0, D // 16)
            def _(k):
                zv[i, pl.ds(k * 16, 16)] = z16

        @pl.loop(0, 2)
        def _(q):
            batch = 2 * c + q
            w = batch * 16 + t

            @pl.loop(0, RS // 16 // 64)
            def _(zb):
                r0 = pl.multiple_of(t * (RS // 16) + zb * 64, 64)
                pltpu.sync_copy(zv, slab.at[pl.ds(r0, 64)])

            plsc.subcore_barrier()

            @pl.loop(0, NCHA)
            def _(ch):
                g0 = batch * 80 + t * NCHA + ch
                cps = [pltpu.async_copy(
                    xs_hbm.at[g0], xsv.at[pl.ds(0, CHA)], sem)]
                for k in range(KN):
                    e0 = pl.multiple_of(
                        ((w * KN + k) * NCHA + ch) * CHP, 8)
                    cps.append(pltpu.async_copy(
                        dst_hbm.at[pl.ds(e0, CHP)], idxs[k], sem))
                for cp in cps:
                    cp.wait()
                for k in range(KN):
                    pltpu.sync_copy(xsv, slab.at[idxs[k]], add=True)

            plsc.subcore_barrier()

            rows = RS // 16
            r0 = pl.multiple_of(t * rows, 8)
            pltpu.sync_copy(slab.at[pl.ds(r0, rows)],
                            out_hbm.at[batch, pl.ds(r0, rows)])
            plsc.subcore_barrier()

    return agg_kernel(xs_grp, dst_flat)


def kernel(x, edge_index, edge_mask, W, b, gamma, beta, prelu_a):
    x2 = x[:, 0, :]
    ei = edge_index[:, :, 1:].astype(jnp.int32)
    mask = edge_mask[:, :, 1:]

    flat_pos = jnp.arange(N * KN, dtype=jnp.int32).reshape(1, N, KN)
    dump = N + (flat_pos % 128)
    dst = jnp.where(mask, ei, dump)

    hist_b = _hist_sc(dst.reshape(-1)).reshape(B, RH * 16)
    hist = hist_b[:, :N].reshape(NT)

    xs = _scale(x2, hist)

    dst_tile = dst.reshape(B * 16, SPT, KN).transpose(0, 2, 1)
    dst_tile = dst_tile.reshape(B * 16, KN, NCHA, CHA)
    pad = jnp.broadcast_to(
        (N + jnp.arange(CHP - CHA, dtype=jnp.int32))[None, None, None, :],
        (B * 16, KN, NCHA, CHP - CHA))
    dst_pad = jnp.concatenate([dst_tile, pad], axis=-1)

    acc = _agg_sc(xs.reshape(B * 16 * NCHA, CHA, D),
                  dst_pad.reshape(-1))

    out = _dense_part(x2.reshape(B, N, D), acc,
                      hist_b.reshape(B, RH * 16, 1),
                      W, b, gamma, beta, prelu_a)
    return out.reshape(NT, D)
